# baseline (speedup 1.0000x reference)
import functools

import jax
import jax.numpy as jnp
from jax import lax
from jax.experimental import pallas as pl
from jax.experimental.pallas import tpu as pltpu
from jax.experimental.pallas import tpu_sc as plsc

N = 10000
D = 128
D2 = D // 2
E = 320000

NC = 2
NS = 16

CB = 128
NCH = 80
NCPS = 2 * NCH
EPAD = NS * NCPS * CB - E
NBUF = 4

NP = 10240
RT = NP // NS
DW = 16

_mesh = plsc.VectorSubcoreMesh(core_axis_name="c", subcore_axis_name="s")
_sc_params = pltpu.CompilerParams(use_tc_tiling_on_sc=False)


@functools.partial(
    pl.kernel,
    mesh=_mesh,
    out_type=jax.ShapeDtypeStruct((NC, NP, DW), jnp.float32),
    compiler_params=_sc_params,
    scratch_types=[
        pltpu.VMEM((NCH, CB), jnp.int32),
        pltpu.VMEM((CB, DW), jnp.float32),
        pltpu.VMEM((CB, DW), jnp.float32),
        pltpu.VMEM_SHARED((NP, DW), jnp.float32),
    ],
)
def _deg_kernel(dstr_hbm, ones_hbm, zeros_hbm, out_hbm,
                dst_v, ones_v, zbuf_v, acc_sh):
    cid = lax.axis_index("c")
    sid = lax.axis_index("s")
    pltpu.sync_copy(dstr_hbm.at[sid, pl.ds(cid * NCH, NCH)], dst_v)
    pltpu.sync_copy(ones_hbm, ones_v)
    pltpu.sync_copy(zeros_hbm, zbuf_v)
    for k in range(RT // CB):
        pltpu.sync_copy(zbuf_v, acc_sh.at[pl.ds(sid * RT + k * CB, CB)])
    plsc.subcore_barrier()

    def body(j, carry):
        pltpu.sync_copy(ones_v, acc_sh.at[dst_v.at[j]], add=True)
        return carry

    lax.fori_loop(0, NCH, body, 0)
    plsc.subcore_barrier()
    for k in range(RT // CB):
        sl = pl.ds(sid * RT + k * CB, CB)
        pltpu.sync_copy(acc_sh.at[sl], out_hbm.at[cid].at[sl])


@functools.partial(
    pl.kernel,
    mesh=_mesh,
    out_type=jax.ShapeDtypeStruct((NP, NC, D2), jnp.float32),
    compiler_params=_sc_params,
    scratch_types=[
        pltpu.VMEM((NCPS, CB), jnp.int32),
        pltpu.VMEM((NCPS, CB), jnp.int32),
        pltpu.VMEM((NBUF, CB, D2), jnp.float32),
        pltpu.VMEM((CB, D2), jnp.float32),
        pltpu.VMEM_SHARED((NP, D2), jnp.float32),
        [pltpu.SemaphoreType.DMA] * NBUF,
    ],
)
def _agg_kernel(p_hbm, srcr_hbm, dstr_hbm, zeros_hbm, out_hbm,
                src_v, dst_v, rows_v, zbuf_v, acc_sh, gsems):
    cid = lax.axis_index("c")
    sid = lax.axis_index("s")
    pltpu.sync_copy(srcr_hbm.at[cid].at[sid], src_v)
    pltpu.sync_copy(dstr_hbm.at[sid], dst_v)
    pltpu.sync_copy(zeros_hbm, zbuf_v)
    for k in range(RT // CB):
        pltpu.sync_copy(zbuf_v, acc_sh.at[pl.ds(sid * RT + k * CB, CB)])
    plsc.subcore_barrier()

    for b in range(NBUF):
        pltpu.async_copy(p_hbm.at[src_v.at[b]], rows_v.at[b], gsems[b])

    def _drain_one(j, b):
        pltpu.make_async_copy(p_hbm.at[src_v.at[j]], rows_v.at[b], gsems[b]).wait()
        pltpu.sync_copy(rows_v.at[b], acc_sh.at[dst_v.at[j]], add=True)

    def outer(g, carry):
        for b in range(NBUF):
            j = g * NBUF + b
            _drain_one(j, b)
            pltpu.async_copy(p_hbm.at[src_v.at[j + NBUF]], rows_v.at[b], gsems[b])
        return carry

    lax.fori_loop(0, NCPS // NBUF - 1, outer, 0)
    for b in range(NBUF):
        _drain_one(NCPS - NBUF + b, b)

    plsc.subcore_barrier()
    for k in range(RT // CB):
        sl = pl.ds(sid * RT + k * CB, CB)
        pltpu.sync_copy(acc_sh.at[sl], out_hbm.at[sl, cid])


BR = 2000


def _dd_from_acc(dacc_ref):
    deg = dacc_ref[0, :, :] + dacc_ref[1, :, :] + 1.0
    return lax.rsqrt(deg)[:, 0:1]


def _pre_body(x_ref, w_ref, dacc_ref, o_ref):
    dd = _dd_from_acc(dacc_ref)
    h = jnp.dot(x_ref[...], w_ref[...], preferred_element_type=jnp.float32)
    o_ref[...] = h * dd


_pre = pl.pallas_call(
    _pre_body,
    grid=(N // BR,),
    in_specs=[
        pl.BlockSpec((BR, D), lambda i: (i, 0)),
        pl.BlockSpec((D, D), lambda i: (0, 0)),
        pl.BlockSpec((2, BR, DW), lambda i: (0, i, 0)),
    ],
    out_specs=pl.BlockSpec((BR, D), lambda i: (i, 0)),
    out_shape=jax.ShapeDtypeStruct((N, D), jnp.float32),
)


def _mid_body(agg_ref, p_ref, dacc_ref, b_ref, w_ref, o_ref):
    dd = _dd_from_acc(dacc_ref)
    z = dd * (agg_ref[...] + p_ref[...]) + b_ref[...]
    h = jnp.maximum(z, 0.0)
    o_ref[...] = jnp.dot(h, w_ref[...], preferred_element_type=jnp.float32) * dd


_mid = pl.pallas_call(
    _mid_body,
    grid=(N // BR,),
    in_specs=[
        pl.BlockSpec((BR, D), lambda i: (i, 0)),
        pl.BlockSpec((BR, D), lambda i: (i, 0)),
        pl.BlockSpec((2, BR, DW), lambda i: (0, i, 0)),
        pl.BlockSpec((1, D), lambda i: (0, 0)),
        pl.BlockSpec((D, D), lambda i: (0, 0)),
    ],
    out_specs=pl.BlockSpec((BR, D), lambda i: (i, 0)),
    out_shape=jax.ShapeDtypeStruct((N, D), jnp.float32),
)


def _post_body(agg_ref, p_ref, dacc_ref, b_ref, o_ref):
    dd = _dd_from_acc(dacc_ref)
    o_ref[...] = dd * (agg_ref[...] + p_ref[...]) + b_ref[...]


_post = pl.pallas_call(
    _post_body,
    grid=(N // BR,),
    in_specs=[
        pl.BlockSpec((BR, D), lambda i: (i, 0)),
        pl.BlockSpec((BR, D), lambda i: (i, 0)),
        pl.BlockSpec((2, BR, DW), lambda i: (0, i, 0)),
        pl.BlockSpec((1, D), lambda i: (0, 0)),
    ],
    out_specs=pl.BlockSpec((BR, D), lambda i: (i, 0)),
    out_shape=jax.ShapeDtypeStruct((N, D), jnp.float32),
)


def kernel(x, edge_index, W1, b1, W2, b2):
    src = edge_index[0].astype(jnp.int32)
    dst = edge_index[1].astype(jnp.int32)
    pad_pos = jnp.arange(EPAD, dtype=jnp.int32)
    pad_src = (pad_pos * 97) % N
    pad_dst = N + pad_pos % (NP - N)
    srcp = jnp.concatenate([src, pad_src])
    src2_r = jnp.stack([2 * srcp, 2 * srcp + 1]).reshape(NC, NS, NCPS, CB)
    dst_r = jnp.concatenate([dst, pad_dst]).reshape(NS, NCPS, CB)

    ones_dw = jnp.ones((CB, DW), jnp.float32)
    zeros_dw = jnp.zeros((CB, DW), jnp.float32)
    zeros_d2 = jnp.zeros((CB, D2), jnp.float32)
    b1r = b1.reshape(1, D)
    b2r = b2.reshape(1, D)

    dacc = _deg_kernel(dst_r, ones_dw, zeros_dw)
    p1 = _pre(x, W1, dacc)
    agg1 = _agg_kernel(p1.reshape(2 * N, D2), src2_r, dst_r, zeros_d2)
    p2 = _mid(agg1.reshape(NP, D), p1, dacc, b1r, W2)
    agg2 = _agg_kernel(p2.reshape(2 * N, D2), src2_r, dst_r, zeros_d2)
    return _post(agg2.reshape(NP, D), p2, dacc, b2r)


# speedup vs baseline: 1.2846x; 1.2846x over previous
import functools

import jax
import jax.numpy as jnp
from jax import lax
from jax.experimental import pallas as pl
from jax.experimental.pallas import tpu as pltpu
from jax.experimental.pallas import tpu_sc as plsc

N = 10000
D = 128
D2 = D // 2
E = 320000

NC = 2
NS = 16

CB = 128
NCH = 80
NCPS = 2 * NCH
EPAD = NS * NCPS * CB - E
NBUF = 4

NP = 10240
RT = NP // NS
DW = 16

_mesh = plsc.VectorSubcoreMesh(core_axis_name="c", subcore_axis_name="s")
_sc_params = pltpu.CompilerParams(use_tc_tiling_on_sc=False)


@functools.partial(
    pl.kernel,
    mesh=_mesh,
    out_type=jax.ShapeDtypeStruct((NC, NP, DW), jnp.float32),
    compiler_params=_sc_params,
    scratch_types=[
        pltpu.VMEM((NCH, CB), jnp.int32),
        pltpu.VMEM((CB, DW), jnp.float32),
        pltpu.VMEM((CB, DW), jnp.float32),
        pltpu.VMEM_SHARED((NP, DW), jnp.float32),
    ],
)
def _deg_kernel(dstr_hbm, ones_hbm, zeros_hbm, out_hbm,
                dst_v, ones_v, zbuf_v, acc_sh):
    cid = lax.axis_index("c")
    sid = lax.axis_index("s")
    pltpu.sync_copy(dstr_hbm.at[sid, pl.ds(cid * NCH, NCH)], dst_v)
    pltpu.sync_copy(ones_hbm, ones_v)
    pltpu.sync_copy(zeros_hbm, zbuf_v)
    for k in range(RT // CB):
        pltpu.sync_copy(zbuf_v, acc_sh.at[pl.ds(sid * RT + k * CB, CB)])
    plsc.subcore_barrier()

    def body(j, carry):
        pltpu.sync_copy(ones_v, acc_sh.at[dst_v.at[j]], add=True)
        return carry

    lax.fori_loop(0, NCH, body, 0)
    plsc.subcore_barrier()
    for k in range(RT // CB):
        sl = pl.ds(sid * RT + k * CB, CB)
        pltpu.sync_copy(acc_sh.at[sl], out_hbm.at[cid].at[sl])


@functools.partial(
    pl.kernel,
    mesh=_mesh,
    out_type=jax.ShapeDtypeStruct((NP, D), jnp.float32),
    compiler_params=_sc_params,
    scratch_types=[
        pltpu.VMEM((NCPS, CB), jnp.int32),
        pltpu.VMEM((NCPS, CB), jnp.int32),
        pltpu.VMEM((NBUF, CB, D2), jnp.float32),
        pltpu.VMEM((CB, D2), jnp.float32),
        pltpu.VMEM_SHARED((NP, D2), jnp.float32),
        [pltpu.SemaphoreType.DMA] * NBUF,
    ],
)
def _agg_kernel(p_hbm, srcr_hbm, dstr_hbm, zeros_hbm, out_hbm,
                src_v, dst_v, rows_v, zbuf_v, acc_sh, gsems):
    cid = lax.axis_index("c")
    sid = lax.axis_index("s")
    pltpu.sync_copy(srcr_hbm.at[cid].at[sid], src_v)
    pltpu.sync_copy(dstr_hbm.at[sid], dst_v)
    pltpu.sync_copy(zeros_hbm, zbuf_v)
    for k in range(RT // CB):
        pltpu.sync_copy(zbuf_v, acc_sh.at[pl.ds(sid * RT + k * CB, CB)])
    plsc.subcore_barrier()

    for b in range(NBUF):
        pltpu.async_copy(p_hbm.at[src_v.at[b]], rows_v.at[b], gsems[b])

    def _drain_one(j, b):
        pltpu.make_async_copy(p_hbm.at[src_v.at[j]], rows_v.at[b], gsems[b]).wait()
        pltpu.sync_copy(rows_v.at[b], acc_sh.at[dst_v.at[j]], add=True)

    def outer(g, carry):
        for b in range(NBUF):
            j = g * NBUF + b
            _drain_one(j, b)
            pltpu.async_copy(p_hbm.at[src_v.at[j + NBUF]], rows_v.at[b], gsems[b])
        return carry

    lax.fori_loop(0, NCPS // NBUF - 1, outer, 0)
    for b in range(NBUF):
        _drain_one(NCPS - NBUF + b, b)

    plsc.subcore_barrier()
    for k in range(RT // CB):
        sl = pl.ds(sid * RT + k * CB, CB)
        pltpu.sync_copy(acc_sh.at[sl], out_hbm.at[sl, pl.ds(cid * D2, D2)])


BR = 2000


def _dd_from_acc(dacc_ref):
    deg = dacc_ref[0, :, :] + dacc_ref[1, :, :] + 1.0
    return lax.rsqrt(deg)[:, 0:1]


def _pre_body(x_ref, w_ref, dacc_ref, o_ref):
    dd = _dd_from_acc(dacc_ref)
    h = jnp.dot(x_ref[...], w_ref[...], preferred_element_type=jnp.float32)
    o_ref[...] = h * dd


_pre = pl.pallas_call(
    _pre_body,
    grid=(N // BR,),
    in_specs=[
        pl.BlockSpec((BR, D), lambda i: (i, 0)),
        pl.BlockSpec((D, D), lambda i: (0, 0)),
        pl.BlockSpec((2, BR, DW), lambda i: (0, i, 0)),
    ],
    out_specs=pl.BlockSpec((BR, D), lambda i: (i, 0)),
    out_shape=jax.ShapeDtypeStruct((N, D), jnp.float32),
)


def _mid_body(agg_ref, p_ref, dacc_ref, b_ref, w_ref, o_ref):
    dd = _dd_from_acc(dacc_ref)
    z = dd * (agg_ref[...] + p_ref[...]) + b_ref[...]
    h = jnp.maximum(z, 0.0)
    o_ref[...] = jnp.dot(h, w_ref[...], preferred_element_type=jnp.float32) * dd


_mid = pl.pallas_call(
    _mid_body,
    grid=(N // BR,),
    in_specs=[
        pl.BlockSpec((BR, D), lambda i: (i, 0)),
        pl.BlockSpec((BR, D), lambda i: (i, 0)),
        pl.BlockSpec((2, BR, DW), lambda i: (0, i, 0)),
        pl.BlockSpec((1, D), lambda i: (0, 0)),
        pl.BlockSpec((D, D), lambda i: (0, 0)),
    ],
    out_specs=pl.BlockSpec((BR, D), lambda i: (i, 0)),
    out_shape=jax.ShapeDtypeStruct((N, D), jnp.float32),
)


def _post_body(agg_ref, p_ref, dacc_ref, b_ref, o_ref):
    dd = _dd_from_acc(dacc_ref)
    o_ref[...] = dd * (agg_ref[...] + p_ref[...]) + b_ref[...]


_post = pl.pallas_call(
    _post_body,
    grid=(N // BR,),
    in_specs=[
        pl.BlockSpec((BR, D), lambda i: (i, 0)),
        pl.BlockSpec((BR, D), lambda i: (i, 0)),
        pl.BlockSpec((2, BR, DW), lambda i: (0, i, 0)),
        pl.BlockSpec((1, D), lambda i: (0, 0)),
    ],
    out_specs=pl.BlockSpec((BR, D), lambda i: (i, 0)),
    out_shape=jax.ShapeDtypeStruct((N, D), jnp.float32),
)


def kernel(x, edge_index, W1, b1, W2, b2):
    src = edge_index[0].astype(jnp.int32)
    dst = edge_index[1].astype(jnp.int32)
    pad_pos = jnp.arange(EPAD, dtype=jnp.int32)
    pad_src = (pad_pos * 97) % N
    pad_dst = N + pad_pos % (NP - N)
    srcp = jnp.concatenate([src, pad_src])
    src2_r = jnp.stack([2 * srcp, 2 * srcp + 1]).reshape(NC, NS, NCPS, CB)
    dst_r = jnp.concatenate([dst, pad_dst]).reshape(NS, NCPS, CB)

    ones_dw = jnp.ones((CB, DW), jnp.float32)
    zeros_dw = jnp.zeros((CB, DW), jnp.float32)
    zeros_d2 = jnp.zeros((CB, D2), jnp.float32)
    b1r = b1.reshape(1, D)
    b2r = b2.reshape(1, D)

    dacc = _deg_kernel(dst_r, ones_dw, zeros_dw)
    p1 = _pre(x, W1, dacc)
    agg1 = _agg_kernel(p1.reshape(2 * N, D2), src2_r, dst_r, zeros_d2)
    p2 = _mid(agg1, p1, dacc, b1r, W2)
    agg2 = _agg_kernel(p2.reshape(2 * N, D2), src2_r, dst_r, zeros_d2)
    return _post(agg2, p2, dacc, b2r)


# speedup vs baseline: 1.3484x; 1.0497x over previous
import functools

import jax
import jax.numpy as jnp
from jax import lax
from jax.experimental import pallas as pl
from jax.experimental.pallas import tpu as pltpu
from jax.experimental.pallas import tpu_sc as plsc

N = 10000
D = 128
D2 = D // 2
E = 320000

NC = 2
NS = 16

CB = 128
NCH = 80
NCPS = 2 * NCH
EPAD = NS * NCPS * CB - E
NBUF = 4

NP = 10240
RT = NP // NS
DW = 16

_mesh = plsc.VectorSubcoreMesh(core_axis_name="c", subcore_axis_name="s")
_sc_params = pltpu.CompilerParams(use_tc_tiling_on_sc=False)


NSB = 4


@functools.partial(
    pl.kernel,
    mesh=_mesh,
    out_type=jax.ShapeDtypeStruct((NP, D), jnp.float32),
    compiler_params=_sc_params,
    scratch_types=[
        pltpu.VMEM((NCH, CB), jnp.int32),
        pltpu.VMEM((CB, DW), jnp.float32),
        pltpu.VMEM((CB, DW), jnp.float32),
        pltpu.VMEM_SHARED((NP, DW), jnp.float32),
        [pltpu.SemaphoreType.DMA] * NSB,
    ],
)
def _deg_kernel(dstr_hbm, ones_hbm, zeros_hbm, out_hbm,
                dst_v, ones_v, zbuf_v, acc_sh, ssems):
    cid = lax.axis_index("c")
    sid = lax.axis_index("s")
    pltpu.sync_copy(dstr_hbm.at[sid, pl.ds(cid * NCH, NCH)], dst_v)
    pltpu.sync_copy(ones_hbm, ones_v)
    pltpu.sync_copy(zeros_hbm, zbuf_v)
    for k in range(RT // CB):
        pltpu.sync_copy(zbuf_v, acc_sh.at[pl.ds(sid * RT + k * CB, CB)])
    plsc.subcore_barrier()

    def _scat(j, s):
        pltpu.async_copy(ones_v, acc_sh.at[dst_v.at[j]], ssems[s], add=True)

    def _wait(s):
        pltpu.make_async_copy(ones_v, acc_sh.at[dst_v.at[0]], ssems[s]).wait()

    for s in range(NSB):
        _scat(s, s)

    def body(g, carry):
        for s in range(NSB):
            _wait(s)
            _scat(g * NSB + s, s)
        return carry

    lax.fori_loop(1, NCH // NSB, body, 0)
    for s in range(NSB):
        _wait(s)
    plsc.subcore_barrier()
    for k in range(RT // CB):
        sl = pl.ds(sid * RT + k * CB, CB)
        pltpu.sync_copy(acc_sh.at[sl], out_hbm.at[sl, pl.ds(cid * DW, DW)])


@functools.partial(
    pl.kernel,
    mesh=_mesh,
    out_type=jax.ShapeDtypeStruct((NP, D), jnp.float32),
    compiler_params=_sc_params,
    scratch_types=[
        pltpu.VMEM((NCPS, CB), jnp.int32),
        pltpu.VMEM((NCPS, CB), jnp.int32),
        pltpu.VMEM((NBUF, CB, D2), jnp.float32),
        pltpu.VMEM((CB, D2), jnp.float32),
        pltpu.VMEM_SHARED((NP, D2), jnp.float32),
        [pltpu.SemaphoreType.DMA] * NBUF,
    ],
)
def _agg_kernel(p_hbm, srcr_hbm, dstr_hbm, zeros_hbm, out_hbm,
                src_v, dst_v, rows_v, zbuf_v, acc_sh, gsems):
    cid = lax.axis_index("c")
    sid = lax.axis_index("s")
    pltpu.sync_copy(srcr_hbm.at[cid].at[sid], src_v)
    pltpu.sync_copy(dstr_hbm.at[sid], dst_v)
    pltpu.sync_copy(zeros_hbm, zbuf_v)
    for k in range(RT // CB):
        pltpu.sync_copy(zbuf_v, acc_sh.at[pl.ds(sid * RT + k * CB, CB)])
    plsc.subcore_barrier()

    for b in range(NBUF):
        pltpu.async_copy(p_hbm.at[src_v.at[b]], rows_v.at[b], gsems[b])

    def _drain_one(j, b):
        pltpu.make_async_copy(p_hbm.at[src_v.at[j]], rows_v.at[b], gsems[b]).wait()
        pltpu.sync_copy(rows_v.at[b], acc_sh.at[dst_v.at[j]], add=True)

    def outer(g, carry):
        for b in range(NBUF):
            j = g * NBUF + b
            _drain_one(j, b)
            pltpu.async_copy(p_hbm.at[src_v.at[j + NBUF]], rows_v.at[b], gsems[b])
        return carry

    lax.fori_loop(0, NCPS // NBUF - 1, outer, 0)
    for b in range(NBUF):
        _drain_one(NCPS - NBUF + b, b)

    plsc.subcore_barrier()
    for k in range(RT // CB):
        sl = pl.ds(sid * RT + k * CB, CB)
        pltpu.sync_copy(acc_sh.at[sl], out_hbm.at[sl, pl.ds(cid * D2, D2)])


BR = 2000


def _dd_from_acc(dacc_ref):
    deg = dacc_ref[:, 0:1] + dacc_ref[:, DW:DW + 1] + 1.0
    return lax.rsqrt(deg)


def _pre_body(x_ref, w_ref, dacc_ref, o_ref):
    dd = _dd_from_acc(dacc_ref)
    h = jnp.dot(x_ref[...], w_ref[...], preferred_element_type=jnp.float32)
    o_ref[...] = h * dd


_pre = pl.pallas_call(
    _pre_body,
    grid=(N // BR,),
    in_specs=[
        pl.BlockSpec((BR, D), lambda i: (i, 0)),
        pl.BlockSpec((D, D), lambda i: (0, 0)),
        pl.BlockSpec((BR, D), lambda i: (i, 0)),
    ],
    out_specs=pl.BlockSpec((BR, D), lambda i: (i, 0)),
    out_shape=jax.ShapeDtypeStruct((N, D), jnp.float32),
)


def _mid_body(agg_ref, p_ref, dacc_ref, b_ref, w_ref, o_ref):
    dd = _dd_from_acc(dacc_ref)
    z = dd * (agg_ref[...] + p_ref[...]) + b_ref[...]
    h = jnp.maximum(z, 0.0)
    o_ref[...] = jnp.dot(h, w_ref[...], preferred_element_type=jnp.float32) * dd


_mid = pl.pallas_call(
    _mid_body,
    grid=(N // BR,),
    in_specs=[
        pl.BlockSpec((BR, D), lambda i: (i, 0)),
        pl.BlockSpec((BR, D), lambda i: (i, 0)),
        pl.BlockSpec((BR, D), lambda i: (i, 0)),
        pl.BlockSpec((1, D), lambda i: (0, 0)),
        pl.BlockSpec((D, D), lambda i: (0, 0)),
    ],
    out_specs=pl.BlockSpec((BR, D), lambda i: (i, 0)),
    out_shape=jax.ShapeDtypeStruct((N, D), jnp.float32),
)


def _post_body(agg_ref, p_ref, dacc_ref, b_ref, o_ref):
    dd = _dd_from_acc(dacc_ref)
    o_ref[...] = dd * (agg_ref[...] + p_ref[...]) + b_ref[...]


_post = pl.pallas_call(
    _post_body,
    grid=(N // BR,),
    in_specs=[
        pl.BlockSpec((BR, D), lambda i: (i, 0)),
        pl.BlockSpec((BR, D), lambda i: (i, 0)),
        pl.BlockSpec((BR, D), lambda i: (i, 0)),
        pl.BlockSpec((1, D), lambda i: (0, 0)),
    ],
    out_specs=pl.BlockSpec((BR, D), lambda i: (i, 0)),
    out_shape=jax.ShapeDtypeStruct((N, D), jnp.float32),
)


def kernel(x, edge_index, W1, b1, W2, b2):
    src = edge_index[0].astype(jnp.int32)
    dst = edge_index[1].astype(jnp.int32)
    pad_pos = jnp.arange(EPAD, dtype=jnp.int32)
    pad_src = (pad_pos * 97) % N
    pad_dst = N + pad_pos % (NP - N)
    srcp = jnp.concatenate([src, pad_src])
    src2_r = jnp.stack([2 * srcp, 2 * srcp + 1]).reshape(NC, NS, NCPS, CB)
    dst_r = jnp.concatenate([dst, pad_dst]).reshape(NS, NCPS, CB)

    ones_dw = jnp.ones((CB, DW), jnp.float32)
    zeros_dw = jnp.zeros((CB, DW), jnp.float32)
    zeros_d2 = jnp.zeros((CB, D2), jnp.float32)
    b1r = b1.reshape(1, D)
    b2r = b2.reshape(1, D)

    dacc = _deg_kernel(dst_r, ones_dw, zeros_dw)
    p1 = _pre(x, W1, dacc)
    agg1 = _agg_kernel(p1.reshape(2 * N, D2), src2_r, dst_r, zeros_d2)
    p2 = _mid(agg1, p1, dacc, b1r, W2)
    agg2 = _agg_kernel(p2.reshape(2 * N, D2), src2_r, dst_r, zeros_d2)
    return _post(agg2, p2, dacc, b2r)
